# baseline (speedup 1.0000x reference)
import jax
import jax.numpy as jnp
from jax.experimental import pallas as pl
from jax.experimental.pallas import tpu as pltpu

_B = 512
_CHUNKS = (1, 4, 8, 9, 5, 1)


def _chunk_sums(p, t):
    t0 = t[:, 0, :, :]
    t1 = t[:, 1, :, :]
    t2 = t[:, 2, :, :]
    d0 = p[:, 0, :, :] - t0
    d1 = p[:, 1, :, :] - t1
    d2 = p[:, 2, :, :] - t2
    e0 = d0 * d0
    s_cell = jnp.where(t0 > 0.5, e0 + 5.0 * (d1 * d1) + d2 * d2, 0.0) \
        + jnp.where(t0 < 0.5, 0.5 * e0, 0.0)
    s = jnp.sum(s_cell, axis=(0, 1)).reshape(1, _B)
    m = jnp.max(jnp.maximum(jnp.maximum(t0, t1), t2), axis=(0, 1)).reshape(1, _B)
    return s, m


def _body(p_hbm, t_hbm, out_ref, *scratch):
    n = len(_CHUNKS)
    bufs = tuple((scratch[2 * i], scratch[2 * i + 1],
                  scratch[2 * n + 2 * i], scratch[2 * n + 2 * i + 1])
                 for i in range(n))
    copies = []
    off = 0
    for (bp, bt, sp, st), sz in zip(bufs, _CHUNKS):
        cp = pltpu.make_async_copy(p_hbm.at[pl.ds(off, sz)], bp, sp)
        ct = pltpu.make_async_copy(t_hbm.at[pl.ds(off, sz)], bt, st)
        copies.append((cp, ct, bp, bt))
        off += sz
    for cp, ct, _, _ in copies:
        cp.start()
        ct.start()
    s_tot = None
    m_tot = None
    for cp, ct, bp, bt in copies:
        cp.wait()
        ct.wait()
        s, m = _chunk_sums(bp[...], bt[...])
        s_tot = s if s_tot is None else s_tot + s
        m_tot = m if m_tot is None else jnp.maximum(m_tot, m)
    sel = m_tot > 0.5
    cnt = jnp.sum(sel.astype(jnp.float32))
    tot = jnp.sum(jnp.where(sel, s_tot, 0.0))
    out_ref[0] = tot / (28.0 * cnt)


def kernel(pred_tensor, target_tensor):
    p = pred_tensor.transpose(1, 3, 2, 0)
    t = target_tensor.transpose(1, 3, 2, 0)
    scratch = []
    for sz in _CHUNKS:
        scratch.append(pltpu.VMEM((sz, 3, 28, _B), jnp.float32))
        scratch.append(pltpu.VMEM((sz, 3, 28, _B), jnp.float32))
    scratch.extend([pltpu.SemaphoreType.DMA] * (2 * len(_CHUNKS)))
    out = pl.pallas_call(
        _body,
        in_specs=[
            pl.BlockSpec(memory_space=pltpu.MemorySpace.HBM),
            pl.BlockSpec(memory_space=pltpu.MemorySpace.HBM),
        ],
        out_specs=pl.BlockSpec(memory_space=pltpu.SMEM),
        out_shape=jax.ShapeDtypeStruct((1,), jnp.float32),
        scratch_shapes=scratch,
    )(p, t)
    return out[0]


# speedup vs baseline: 1.0278x; 1.0278x over previous
import jax
import jax.numpy as jnp
from jax.experimental import pallas as pl
from jax.experimental.pallas import tpu as pltpu

_B = 512
_CHUNKS = (1, 3, 6, 8, 6, 3, 1)


def _chunk_sums(p, t):
    t0 = t[:, 0, :, :]
    t1 = t[:, 1, :, :]
    t2 = t[:, 2, :, :]
    d0 = p[:, 0, :, :] - t0
    d1 = p[:, 1, :, :] - t1
    d2 = p[:, 2, :, :] - t2
    e0 = d0 * d0
    s_cell = jnp.where(t0 > 0.5, e0 + 5.0 * (d1 * d1) + d2 * d2, 0.0) \
        + jnp.where(t0 < 0.5, 0.5 * e0, 0.0)
    s = jnp.sum(s_cell, axis=(0, 1)).reshape(1, _B)
    m = jnp.max(jnp.maximum(jnp.maximum(t0, t1), t2), axis=(0, 1)).reshape(1, _B)
    return s, m


def _body(p_hbm, t_hbm, out_ref, *scratch):
    n = len(_CHUNKS)
    bufs = tuple((scratch[2 * i], scratch[2 * i + 1],
                  scratch[2 * n + 2 * i], scratch[2 * n + 2 * i + 1])
                 for i in range(n))
    copies = []
    off = 0
    for (bp, bt, sp, st), sz in zip(bufs, _CHUNKS):
        cp = pltpu.make_async_copy(p_hbm.at[pl.ds(off, sz)], bp, sp)
        ct = pltpu.make_async_copy(t_hbm.at[pl.ds(off, sz)], bt, st)
        copies.append((cp, ct, bp, bt))
        off += sz
    for cp, ct, _, _ in copies:
        cp.start()
        ct.start()
    s_tot = None
    m_tot = None
    for cp, ct, bp, bt in copies:
        cp.wait()
        ct.wait()
        s, m = _chunk_sums(bp[...], bt[...])
        s_tot = s if s_tot is None else s_tot + s
        m_tot = m if m_tot is None else jnp.maximum(m_tot, m)
    sel = m_tot > 0.5
    cnt = jnp.sum(sel.astype(jnp.float32))
    tot = jnp.sum(jnp.where(sel, s_tot, 0.0))
    out_ref[0] = tot / (28.0 * cnt)


def kernel(pred_tensor, target_tensor):
    p = pred_tensor.transpose(1, 3, 2, 0)
    t = target_tensor.transpose(1, 3, 2, 0)
    scratch = []
    for sz in _CHUNKS:
        scratch.append(pltpu.VMEM((sz, 3, 28, _B), jnp.float32))
        scratch.append(pltpu.VMEM((sz, 3, 28, _B), jnp.float32))
    scratch.extend([pltpu.SemaphoreType.DMA] * (2 * len(_CHUNKS)))
    out = pl.pallas_call(
        _body,
        in_specs=[
            pl.BlockSpec(memory_space=pltpu.MemorySpace.HBM),
            pl.BlockSpec(memory_space=pltpu.MemorySpace.HBM),
        ],
        out_specs=pl.BlockSpec(memory_space=pltpu.SMEM),
        out_shape=jax.ShapeDtypeStruct((1,), jnp.float32),
        scratch_shapes=scratch,
    )(p, t)
    return out[0]


# speedup vs baseline: 1.0472x; 1.0189x over previous
import jax
import jax.numpy as jnp
from jax.experimental import pallas as pl
from jax.experimental.pallas import tpu as pltpu

_B = 512
_CHUNKS = (2, 5, 7, 7, 5, 2)


def _chunk_sums(p, t):
    t0 = t[:, 0, :, :]
    t1 = t[:, 1, :, :]
    t2 = t[:, 2, :, :]
    d0 = p[:, 0, :, :] - t0
    d1 = p[:, 1, :, :] - t1
    d2 = p[:, 2, :, :] - t2
    e0 = d0 * d0
    s_cell = jnp.where(t0 > 0.5, e0 + 5.0 * (d1 * d1) + d2 * d2, 0.0) \
        + jnp.where(t0 < 0.5, 0.5 * e0, 0.0)
    s = jnp.sum(s_cell, axis=(0, 1)).reshape(1, _B)
    m = jnp.max(jnp.maximum(jnp.maximum(t0, t1), t2), axis=(0, 1)).reshape(1, _B)
    return s, m


def _body(p_hbm, t_hbm, out_ref, *scratch):
    n = len(_CHUNKS)
    bufs = tuple((scratch[2 * i], scratch[2 * i + 1],
                  scratch[2 * n + 2 * i], scratch[2 * n + 2 * i + 1])
                 for i in range(n))
    copies = []
    off = 0
    for (bp, bt, sp, st), sz in zip(bufs, _CHUNKS):
        cp = pltpu.make_async_copy(p_hbm.at[pl.ds(off, sz)], bp, sp)
        ct = pltpu.make_async_copy(t_hbm.at[pl.ds(off, sz)], bt, st)
        copies.append((cp, ct, bp, bt))
        off += sz
    for cp, ct, _, _ in copies:
        cp.start()
        ct.start()
    s_tot = None
    m_tot = None
    for cp, ct, bp, bt in copies:
        cp.wait()
        ct.wait()
        s, m = _chunk_sums(bp[...], bt[...])
        s_tot = s if s_tot is None else s_tot + s
        m_tot = m if m_tot is None else jnp.maximum(m_tot, m)
    sel = m_tot > 0.5
    cnt = jnp.sum(sel.astype(jnp.float32))
    tot = jnp.sum(jnp.where(sel, s_tot, 0.0))
    out_ref[0] = tot / (28.0 * cnt)


def kernel(pred_tensor, target_tensor):
    p = pred_tensor.transpose(1, 3, 2, 0)
    t = target_tensor.transpose(1, 3, 2, 0)
    scratch = []
    for sz in _CHUNKS:
        scratch.append(pltpu.VMEM((sz, 3, 28, _B), jnp.float32))
        scratch.append(pltpu.VMEM((sz, 3, 28, _B), jnp.float32))
    scratch.extend([pltpu.SemaphoreType.DMA] * (2 * len(_CHUNKS)))
    out = pl.pallas_call(
        _body,
        in_specs=[
            pl.BlockSpec(memory_space=pltpu.MemorySpace.HBM),
            pl.BlockSpec(memory_space=pltpu.MemorySpace.HBM),
        ],
        out_specs=pl.BlockSpec(memory_space=pltpu.SMEM),
        out_shape=jax.ShapeDtypeStruct((1,), jnp.float32),
        scratch_shapes=scratch,
    )(p, t)
    return out[0]
